# baseline (speedup 1.0000x reference)
import jax
import jax.numpy as jnp
from jax.experimental import pallas as pl
from jax.experimental.pallas import tpu as pltpu

_B, _S, _D, _H, _O, _SKP, _E = 8, 512, 256, 256, 256, 256, 8
F32 = jnp.float32


def _gating_kernel(spk_ref, wgt_ref, bg_ref, gates_ref, loss_ref):
    logits = jnp.dot(spk_ref[...], wgt_ref[...], preferred_element_type=F32) + bg_ref[...]
    col = jax.lax.broadcasted_iota(jnp.int32, (_B, _E), 1)
    m1 = jnp.max(logits, axis=1, keepdims=True)
    i1 = jnp.min(jnp.where(logits == m1, col, _E), axis=1, keepdims=True)
    masked = jnp.where(col == i1, -jnp.inf, logits)
    m2 = jnp.max(masked, axis=1, keepdims=True)
    i2 = jnp.min(jnp.where(masked == m2, col, _E), axis=1, keepdims=True)
    e2 = jnp.exp(m2 - m1)
    denom = 1.0 + e2
    gates = jnp.where(col == i1, 1.0 / denom, jnp.where(col == i2, e2 / denom, 0.0))
    gates_ref[...] = gates
    importance = jnp.sum(gates, axis=0, keepdims=True)
    load = jnp.sum((gates > 0.0).astype(F32), axis=0, keepdims=True)

    def cv2(v):
        mean = jnp.mean(v, axis=1, keepdims=True)
        var = jnp.sum((v - mean) ** 2, axis=1, keepdims=True) / (_E - 1)
        return var / (mean * mean + 1e-10)

    loss_ref[...] = (cv2(importance) + cv2(load)) * 0.01


def _lstm_kernel(x2_ref, wih_ref, whh_ref, b_ref, gcol_ref, wfc_ref, bfc_ref,
                 out_ref, xw_ref, acc_ref):
    e = pl.program_id(0)

    xw_ref[...] = (
        jnp.dot(x2_ref[...], wih_ref[0], preferred_element_type=F32) + b_ref[0]
    )

    @pl.when(e == 0)
    def _():
        acc_ref[...] = jnp.zeros_like(acc_ref)

    whh = whh_ref[0]
    ge = gcol_ref[0]

    def step(t, carry):
        h, c = carry
        sl = pl.ds(t * _B, _B)
        z = xw_ref[sl, :] + jnp.dot(
            h.astype(jnp.bfloat16), whh, preferred_element_type=F32)
        i = jax.nn.sigmoid(z[:, :_H])
        f = jax.nn.sigmoid(z[:, _H:2 * _H])
        g = jnp.tanh(z[:, 2 * _H:3 * _H])
        o = jax.nn.sigmoid(z[:, 3 * _H:])
        c = f * c + i * g
        h = o * jnp.tanh(c)
        acc_ref[sl, :] = acc_ref[sl, :] + ge * h
        return (h, c)

    zeros = jnp.zeros((_B, _H), F32)
    jax.lax.fori_loop(0, _S, step, (zeros, zeros), unroll=2)

    @pl.when(e == _E - 1)
    def _():
        out_ref[...] = (
            jnp.dot(acc_ref[...], wfc_ref[...], preferred_element_type=F32)
            + bfc_ref[...]
        )


def kernel(x, spk, Wg, bg, W_ih, W_hh, b_ih, b_hh, W_fc, b_fc):
    x2 = jnp.swapaxes(x, 0, 1).reshape(_S * _B, _D)
    W_ihT = jnp.swapaxes(W_ih, 1, 2)
    W_hhT = jnp.swapaxes(W_hh, 1, 2).astype(jnp.bfloat16)
    b = (b_ih + b_hh).reshape(_E, 1, 4 * _H)
    W_fcT = W_fc.T
    bfc2 = b_fc.reshape(1, _O)
    bg2 = bg.reshape(1, _E)
    WgT = Wg.T

    gates, loss2 = pl.pallas_call(
        _gating_kernel,
        out_shape=(
            jax.ShapeDtypeStruct((_B, _E), F32),
            jax.ShapeDtypeStruct((1, 1), F32),
        ),
    )(spk, WgT, bg2)

    gcol = gates.T.reshape(_E, _B, 1)

    out2 = pl.pallas_call(
        _lstm_kernel,
        grid=(_E,),
        in_specs=[
            pl.BlockSpec((_S * _B, _D), lambda e: (0, 0)),
            pl.BlockSpec((1, _D, 4 * _H), lambda e: (e, 0, 0)),
            pl.BlockSpec((1, _H, 4 * _H), lambda e: (e, 0, 0)),
            pl.BlockSpec((1, 1, 4 * _H), lambda e: (e, 0, 0)),
            pl.BlockSpec((1, _B, 1), lambda e: (e, 0, 0)),
            pl.BlockSpec((_H, _O), lambda e: (0, 0)),
            pl.BlockSpec((1, _O), lambda e: (0, 0)),
        ],
        out_specs=pl.BlockSpec((_S * _B, _O), lambda e: (0, 0)),
        out_shape=jax.ShapeDtypeStruct((_S * _B, _O), F32),
        scratch_shapes=[
            pltpu.VMEM((_S * _B, 4 * _H), F32),
            pltpu.VMEM((_S * _B, _H), F32),
        ],
    )(x2, W_ihT, W_hhT, b, gcol, W_fcT, bfc2)

    out = out2.reshape(_S, _B, _O).swapaxes(0, 1)
    return (out, loss2[0, 0])


# speedup vs baseline: 2.0640x; 2.0640x over previous
import jax
import jax.numpy as jnp
from jax.experimental import pallas as pl
from jax.experimental.pallas import tpu as pltpu

_B, _S, _D, _H, _O, _SKP, _E = 8, 512, 256, 256, 256, 256, 8
_TS = 64
_NCHUNK = _S // _TS
F32 = jnp.float32
BF16 = jnp.bfloat16


def _gating_kernel(spk_ref, wgt_ref, bg_ref, gates_ref, loss_ref):
    logits = jnp.dot(spk_ref[...], wgt_ref[...], preferred_element_type=F32) + bg_ref[...]
    col = jax.lax.broadcasted_iota(jnp.int32, (_B, _E), 1)
    m1 = jnp.max(logits, axis=1, keepdims=True)
    i1 = jnp.min(jnp.where(logits == m1, col, _E), axis=1, keepdims=True)
    masked = jnp.where(col == i1, -jnp.inf, logits)
    m2 = jnp.max(masked, axis=1, keepdims=True)
    i2 = jnp.min(jnp.where(masked == m2, col, _E), axis=1, keepdims=True)
    e2 = jnp.exp(m2 - m1)
    denom = 1.0 + e2
    gates = jnp.where(col == i1, 1.0 / denom, jnp.where(col == i2, e2 / denom, 0.0))
    gates_ref[...] = gates
    importance = jnp.sum(gates, axis=0, keepdims=True)
    load = jnp.sum((gates > 0.0).astype(F32), axis=0, keepdims=True)

    def cv2(v):
        mean = jnp.mean(v, axis=1, keepdims=True)
        var = jnp.sum((v - mean) ** 2, axis=1, keepdims=True) / (_E - 1)
        return var / (mean * mean + 1e-10)

    loss_ref[...] = (cv2(importance) + cv2(load)) * 0.01


def _lstm_kernel(x2_ref, wih_ref, whh_ref, b_ref, gcol_ref, wfc_ref, bfc_ref,
                 out_ref, xw_ref, comb_ref, h_ref, c_ref):
    t0 = pl.program_id(0)

    xc = x2_ref[...].astype(BF16)
    for e in range(_E):
        xw_ref[e] = (
            jnp.dot(xc, wih_ref[e], preferred_element_type=F32) + b_ref[e]
        )

    @pl.when(t0 == 0)
    def _():
        h_ref[...] = jnp.zeros_like(h_ref)
        c_ref[...] = jnp.zeros_like(c_ref)

    whh = [whh_ref[e] for e in range(_E)]
    ge = [gcol_ref[e] for e in range(_E)]

    h0 = h_ref[...]
    c0 = c_ref[...]

    def step(t, carry):
        h_all, c_all = carry
        sl = pl.ds(t * _B, _B)
        comb = None
        new_h, new_c = [], []
        for e in range(_E):
            he = h_all[e * _B:(e + 1) * _B, :]
            ce = c_all[e * _B:(e + 1) * _B, :]
            z = xw_ref[e, sl, :] + jnp.dot(
                he.astype(BF16), whh[e], preferred_element_type=F32)
            i = jax.nn.sigmoid(z[:, :_H])
            f = jax.nn.sigmoid(z[:, _H:2 * _H])
            g = jnp.tanh(z[:, 2 * _H:3 * _H])
            o = jax.nn.sigmoid(z[:, 3 * _H:])
            ce = f * ce + i * g
            he = o * jnp.tanh(ce)
            new_h.append(he)
            new_c.append(ce)
            contrib = ge[e] * he
            comb = contrib if comb is None else comb + contrib
        comb_ref[sl, :] = comb
        return (jnp.concatenate(new_h, axis=0), jnp.concatenate(new_c, axis=0))

    h_fin, c_fin = jax.lax.fori_loop(0, _TS, step, (h0, c0))
    h_ref[...] = h_fin
    c_ref[...] = c_fin

    out_ref[...] = (
        jnp.dot(comb_ref[...].astype(BF16), wfc_ref[...],
                preferred_element_type=F32)
        + bfc_ref[...]
    )


def kernel(x, spk, Wg, bg, W_ih, W_hh, b_ih, b_hh, W_fc, b_fc):
    x2 = jnp.swapaxes(x, 0, 1).reshape(_S * _B, _D)
    W_ihT = jnp.swapaxes(W_ih, 1, 2).astype(BF16)
    W_hhT = jnp.swapaxes(W_hh, 1, 2).astype(BF16)
    b = (b_ih + b_hh).reshape(_E, 1, 4 * _H)
    W_fcT = W_fc.T.astype(BF16)
    bfc2 = b_fc.reshape(1, _O)
    bg2 = bg.reshape(1, _E)
    WgT = Wg.T

    gates, loss2 = pl.pallas_call(
        _gating_kernel,
        out_shape=(
            jax.ShapeDtypeStruct((_B, _E), F32),
            jax.ShapeDtypeStruct((1, 1), F32),
        ),
    )(spk, WgT, bg2)

    gcol = gates.T.reshape(_E, _B, 1)

    out2 = pl.pallas_call(
        _lstm_kernel,
        grid=(_NCHUNK,),
        in_specs=[
            pl.BlockSpec((_TS * _B, _D), lambda t: (t, 0)),
            pl.BlockSpec((_E, _D, 4 * _H), lambda t: (0, 0, 0)),
            pl.BlockSpec((_E, _H, 4 * _H), lambda t: (0, 0, 0)),
            pl.BlockSpec((_E, 1, 4 * _H), lambda t: (0, 0, 0)),
            pl.BlockSpec((_E, _B, 1), lambda t: (0, 0, 0)),
            pl.BlockSpec((_H, _O), lambda t: (0, 0)),
            pl.BlockSpec((1, _O), lambda t: (0, 0)),
        ],
        out_specs=pl.BlockSpec((_TS * _B, _O), lambda t: (t, 0)),
        out_shape=jax.ShapeDtypeStruct((_S * _B, _O), F32),
        scratch_shapes=[
            pltpu.VMEM((_E, _TS * _B, 4 * _H), F32),
            pltpu.VMEM((_TS * _B, _H), F32),
            pltpu.VMEM((_E * _B, _H), F32),
            pltpu.VMEM((_E * _B, _H), F32),
        ],
    )(x2, W_ihT, W_hhT, b, gcol, W_fcT, bfc2)

    out = out2.reshape(_S, _B, _O).swapaxes(0, 1)
    return (out, loss2[0, 0])


# speedup vs baseline: 2.0882x; 1.0117x over previous
import jax
import jax.numpy as jnp
from jax.experimental import pallas as pl
from jax.experimental.pallas import tpu as pltpu

_B, _S, _D, _H, _O, _SKP, _E = 8, 512, 256, 256, 256, 256, 8
_TS = 64
_NCHUNK = _S // _TS
F32 = jnp.float32
BF16 = jnp.bfloat16


def _gating_kernel(spk_ref, wgt_ref, bg_ref, gates_ref, loss_ref):
    logits = jnp.dot(spk_ref[...], wgt_ref[...], preferred_element_type=F32) + bg_ref[...]
    col = jax.lax.broadcasted_iota(jnp.int32, (_B, _E), 1)
    m1 = jnp.max(logits, axis=1, keepdims=True)
    i1 = jnp.min(jnp.where(logits == m1, col, _E), axis=1, keepdims=True)
    masked = jnp.where(col == i1, -jnp.inf, logits)
    m2 = jnp.max(masked, axis=1, keepdims=True)
    i2 = jnp.min(jnp.where(masked == m2, col, _E), axis=1, keepdims=True)
    e2 = jnp.exp(m2 - m1)
    denom = 1.0 + e2
    gates = jnp.where(col == i1, 1.0 / denom, jnp.where(col == i2, e2 / denom, 0.0))
    gates_ref[...] = gates
    importance = jnp.sum(gates, axis=0, keepdims=True)
    load = jnp.sum((gates > 0.0).astype(F32), axis=0, keepdims=True)

    def cv2(v):
        mean = jnp.mean(v, axis=1, keepdims=True)
        var = jnp.sum((v - mean) ** 2, axis=1, keepdims=True) / (_E - 1)
        return var / (mean * mean + 1e-10)

    loss_ref[...] = (cv2(importance) + cv2(load)) * 0.01


def _lstm_kernel(x2_ref, wih_ref, whh_ref, b_ref, gcol_ref, wfc_ref, bfc_ref,
                 out_ref, xw_ref, comb_ref, h_ref, c_ref):
    t0 = pl.program_id(0)

    xc = x2_ref[...].astype(BF16)
    for e in range(_E):
        xw_ref[e] = (
            jnp.dot(xc, wih_ref[e], preferred_element_type=F32) + b_ref[e]
        )

    @pl.when(t0 == 0)
    def _():
        h_ref[...] = jnp.zeros_like(h_ref)
        c_ref[...] = jnp.zeros_like(c_ref)

    ge = [gcol_ref[e] for e in range(_E)]

    h0 = tuple(h_ref[e * _B:(e + 1) * _B, :] for e in range(_E))
    c0 = tuple(c_ref[e * _B:(e + 1) * _B, :] for e in range(_E))

    def step(t, carry):
        hs, cs = carry
        sl = pl.ds(t * _B, _B)
        comb = None
        new_h, new_c = [], []
        for e in range(_E):
            z = xw_ref[e, sl, :] + jnp.dot(
                hs[e].astype(BF16), whh_ref[e], preferred_element_type=F32)
            i = jax.nn.sigmoid(z[:, :_H])
            f = jax.nn.sigmoid(z[:, _H:2 * _H])
            g = jnp.tanh(z[:, 2 * _H:3 * _H])
            o = jax.nn.sigmoid(z[:, 3 * _H:])
            ce = f * cs[e] + i * g
            he = o * jnp.tanh(ce)
            new_h.append(he)
            new_c.append(ce)
            contrib = ge[e] * he
            comb = contrib if comb is None else comb + contrib
        comb_ref[sl, :] = comb
        return (tuple(new_h), tuple(new_c))

    hs_fin, cs_fin = jax.lax.fori_loop(0, _TS, step, (h0, c0))
    for e in range(_E):
        h_ref[e * _B:(e + 1) * _B, :] = hs_fin[e]
        c_ref[e * _B:(e + 1) * _B, :] = cs_fin[e]

    out_ref[...] = (
        jnp.dot(comb_ref[...].astype(BF16), wfc_ref[...],
                preferred_element_type=F32)
        + bfc_ref[...]
    )


def kernel(x, spk, Wg, bg, W_ih, W_hh, b_ih, b_hh, W_fc, b_fc):
    x2 = jnp.swapaxes(x, 0, 1).reshape(_S * _B, _D)
    W_ihT = jnp.swapaxes(W_ih, 1, 2).astype(BF16)
    W_hhT = jnp.swapaxes(W_hh, 1, 2).astype(BF16)
    b = (b_ih + b_hh).reshape(_E, 1, 4 * _H)
    W_fcT = W_fc.T.astype(BF16)
    bfc2 = b_fc.reshape(1, _O)
    bg2 = bg.reshape(1, _E)
    WgT = Wg.T

    gates, loss2 = pl.pallas_call(
        _gating_kernel,
        out_shape=(
            jax.ShapeDtypeStruct((_B, _E), F32),
            jax.ShapeDtypeStruct((1, 1), F32),
        ),
    )(spk, WgT, bg2)

    gcol = gates.T.reshape(_E, _B, 1)

    out2 = pl.pallas_call(
        _lstm_kernel,
        grid=(_NCHUNK,),
        in_specs=[
            pl.BlockSpec((_TS * _B, _D), lambda t: (t, 0)),
            pl.BlockSpec((_E, _D, 4 * _H), lambda t: (0, 0, 0)),
            pl.BlockSpec((_E, _H, 4 * _H), lambda t: (0, 0, 0)),
            pl.BlockSpec((_E, 1, 4 * _H), lambda t: (0, 0, 0)),
            pl.BlockSpec((_E, _B, 1), lambda t: (0, 0, 0)),
            pl.BlockSpec((_H, _O), lambda t: (0, 0)),
            pl.BlockSpec((1, _O), lambda t: (0, 0)),
        ],
        out_specs=pl.BlockSpec((_TS * _B, _O), lambda t: (t, 0)),
        out_shape=jax.ShapeDtypeStruct((_S * _B, _O), F32),
        scratch_shapes=[
            pltpu.VMEM((_E, _TS * _B, 4 * _H), F32),
            pltpu.VMEM((_TS * _B, _H), F32),
            pltpu.VMEM((_E * _B, _H), F32),
            pltpu.VMEM((_E * _B, _H), F32),
        ],
    )(x2, W_ihT, W_hhT, b, gcol, W_fcT, bfc2)

    out = out2.reshape(_S, _B, _O).swapaxes(0, 1)
    return (out, loss2[0, 0])


# speedup vs baseline: 2.2865x; 1.0950x over previous
import jax
import jax.numpy as jnp
from jax.experimental import pallas as pl
from jax.experimental.pallas import tpu as pltpu

_B, _S, _D, _H, _O, _SKP, _E = 8, 512, 256, 256, 256, 256, 8
_TS = 64
_NCHUNK = _S // _TS
F32 = jnp.float32
BF16 = jnp.bfloat16


def _gating_kernel(spk_ref, wgt_ref, bg_ref, gates_ref, loss_ref):
    logits = jnp.dot(spk_ref[...], wgt_ref[...], preferred_element_type=F32) + bg_ref[...]
    col = jax.lax.broadcasted_iota(jnp.int32, (_B, _E), 1)
    m1 = jnp.max(logits, axis=1, keepdims=True)
    i1 = jnp.min(jnp.where(logits == m1, col, _E), axis=1, keepdims=True)
    masked = jnp.where(col == i1, -jnp.inf, logits)
    m2 = jnp.max(masked, axis=1, keepdims=True)
    i2 = jnp.min(jnp.where(masked == m2, col, _E), axis=1, keepdims=True)
    e2 = jnp.exp(m2 - m1)
    denom = 1.0 + e2
    gates = jnp.where(col == i1, 1.0 / denom, jnp.where(col == i2, e2 / denom, 0.0))
    gates_ref[...] = gates
    importance = jnp.sum(gates, axis=0, keepdims=True)
    load = jnp.sum((gates > 0.0).astype(F32), axis=0, keepdims=True)

    def cv2(v):
        mean = jnp.mean(v, axis=1, keepdims=True)
        var = jnp.sum((v - mean) ** 2, axis=1, keepdims=True) / (_E - 1)
        return var / (mean * mean + 1e-10)

    loss_ref[...] = (cv2(importance) + cv2(load)) * 0.01


def _lstm_kernel(x2_ref, wih_ref, whh_ref, b_ref, gcol_ref, wfc_ref, bfc_ref,
                 out_ref, xw_ref, comb_ref, h_ref, c_ref):
    t0 = pl.program_id(0)

    xc = x2_ref[...].astype(BF16)
    for e in range(_E):
        xw_ref[e] = (
            jnp.dot(xc, wih_ref[e], preferred_element_type=F32) + b_ref[e]
        )

    @pl.when(t0 == 0)
    def _():
        h_ref[...] = jnp.zeros_like(h_ref)
        c_ref[...] = jnp.zeros_like(c_ref)

    ge = [gcol_ref[e] for e in range(_E)]

    h0 = tuple(h_ref[e * _B:(e + 1) * _B, :] for e in range(_E))
    c0 = tuple(c_ref[e * _B:(e + 1) * _B, :] for e in range(_E))

    def step(t, carry):
        hs, cs = carry
        sl = pl.ds(t * _B, _B)
        comb = None
        new_h, new_c = [], []
        for e in range(_E):
            z = xw_ref[e, sl, :] + jnp.dot(
                hs[e].astype(BF16), whh_ref[e], preferred_element_type=F32)
            i = jax.nn.sigmoid(z[:, :_H])
            f = jax.nn.sigmoid(z[:, _H:2 * _H])
            g = jnp.tanh(z[:, 2 * _H:3 * _H])
            o = jax.nn.sigmoid(z[:, 3 * _H:])
            ce = f * cs[e] + i * g
            he = o * jnp.tanh(ce)
            new_h.append(he)
            new_c.append(ce)
            contrib = ge[e] * he
            comb = contrib if comb is None else comb + contrib
        comb_ref[sl, :] = comb
        return (tuple(new_h), tuple(new_c))

    hs_fin, cs_fin = jax.lax.fori_loop(0, _TS, step, (h0, c0), unroll=2)
    for e in range(_E):
        h_ref[e * _B:(e + 1) * _B, :] = hs_fin[e]
        c_ref[e * _B:(e + 1) * _B, :] = cs_fin[e]

    out_ref[...] = (
        jnp.dot(comb_ref[...].astype(BF16), wfc_ref[...],
                preferred_element_type=F32)
        + bfc_ref[...]
    )


def kernel(x, spk, Wg, bg, W_ih, W_hh, b_ih, b_hh, W_fc, b_fc):
    x2 = jnp.swapaxes(x, 0, 1).reshape(_S * _B, _D)
    W_ihT = jnp.swapaxes(W_ih, 1, 2).astype(BF16)
    W_hhT = jnp.swapaxes(W_hh, 1, 2).astype(BF16)
    b = (b_ih + b_hh).reshape(_E, 1, 4 * _H)
    W_fcT = W_fc.T.astype(BF16)
    bfc2 = b_fc.reshape(1, _O)
    bg2 = bg.reshape(1, _E)
    WgT = Wg.T

    gates, loss2 = pl.pallas_call(
        _gating_kernel,
        out_shape=(
            jax.ShapeDtypeStruct((_B, _E), F32),
            jax.ShapeDtypeStruct((1, 1), F32),
        ),
    )(spk, WgT, bg2)

    gcol = gates.T.reshape(_E, _B, 1)

    out2 = pl.pallas_call(
        _lstm_kernel,
        grid=(_NCHUNK,),
        in_specs=[
            pl.BlockSpec((_TS * _B, _D), lambda t: (t, 0)),
            pl.BlockSpec((_E, _D, 4 * _H), lambda t: (0, 0, 0)),
            pl.BlockSpec((_E, _H, 4 * _H), lambda t: (0, 0, 0)),
            pl.BlockSpec((_E, 1, 4 * _H), lambda t: (0, 0, 0)),
            pl.BlockSpec((_E, _B, 1), lambda t: (0, 0, 0)),
            pl.BlockSpec((_H, _O), lambda t: (0, 0)),
            pl.BlockSpec((1, _O), lambda t: (0, 0)),
        ],
        out_specs=pl.BlockSpec((_TS * _B, _O), lambda t: (t, 0)),
        out_shape=jax.ShapeDtypeStruct((_S * _B, _O), F32),
        scratch_shapes=[
            pltpu.VMEM((_E, _TS * _B, 4 * _H), F32),
            pltpu.VMEM((_TS * _B, _H), F32),
            pltpu.VMEM((_E * _B, _H), F32),
            pltpu.VMEM((_E * _B, _H), F32),
        ],
    )(x2, W_ihT, W_hhT, b, gcol, W_fcT, bfc2)

    out = out2.reshape(_S, _B, _O).swapaxes(0, 1)
    return (out, loss2[0, 0])


# speedup vs baseline: 2.4040x; 1.0514x over previous
import jax
import jax.numpy as jnp
from jax.experimental import pallas as pl
from jax.experimental.pallas import tpu as pltpu

_B, _S, _D, _H, _O, _SKP, _E = 8, 512, 256, 256, 256, 256, 8
_TS = 64
_NCHUNK = _S // _TS
F32 = jnp.float32
BF16 = jnp.bfloat16


def _gating_kernel(spk_ref, wgt_ref, bg_ref, gates_ref, loss_ref):
    logits = jnp.dot(spk_ref[...], wgt_ref[...], preferred_element_type=F32) + bg_ref[...]
    col = jax.lax.broadcasted_iota(jnp.int32, (_B, _E), 1)
    m1 = jnp.max(logits, axis=1, keepdims=True)
    i1 = jnp.min(jnp.where(logits == m1, col, _E), axis=1, keepdims=True)
    masked = jnp.where(col == i1, -jnp.inf, logits)
    m2 = jnp.max(masked, axis=1, keepdims=True)
    i2 = jnp.min(jnp.where(masked == m2, col, _E), axis=1, keepdims=True)
    e2 = jnp.exp(m2 - m1)
    denom = 1.0 + e2
    gates = jnp.where(col == i1, 1.0 / denom, jnp.where(col == i2, e2 / denom, 0.0))
    gates_ref[...] = gates
    importance = jnp.sum(gates, axis=0, keepdims=True)
    load = jnp.sum((gates > 0.0).astype(F32), axis=0, keepdims=True)

    def cv2(v):
        mean = jnp.mean(v, axis=1, keepdims=True)
        var = jnp.sum((v - mean) ** 2, axis=1, keepdims=True) / (_E - 1)
        return var / (mean * mean + 1e-10)

    loss_ref[...] = (cv2(importance) + cv2(load)) * 0.01


def _lstm_kernel(x2_ref, wih_ref, whh_ref, b_ref, gcol_ref, wfc_ref, bfc_ref,
                 out_ref, xw_ref, comb_ref, h_ref, c_ref):
    t0 = pl.program_id(0)

    xc = x2_ref[...].astype(BF16)
    for e in range(_E):
        xw_ref[e] = (
            jnp.dot(xc, wih_ref[e], preferred_element_type=F32) + b_ref[e]
        )

    @pl.when(t0 == 0)
    def _():
        h_ref[...] = jnp.zeros_like(h_ref)
        c_ref[...] = jnp.zeros_like(c_ref)

    ge = [gcol_ref[e] for e in range(_E)]

    h0 = tuple(h_ref[e * _B:(e + 1) * _B, :] for e in range(_E))
    c0 = tuple(c_ref[e * _B:(e + 1) * _B, :] for e in range(_E))

    def step(t, carry):
        hs, cs = carry
        sl = pl.ds(t * _B, _B)
        comb = None
        new_h, new_c = [], []
        for e in range(_E):
            z = xw_ref[e, sl, :] + jnp.dot(
                hs[e].astype(BF16), whh_ref[e], preferred_element_type=F32)
            i = jax.nn.sigmoid(z[:, :_H])
            f = jax.nn.sigmoid(z[:, _H:2 * _H])
            g = jnp.tanh(z[:, 2 * _H:3 * _H])
            o = jax.nn.sigmoid(z[:, 3 * _H:])
            ce = f * cs[e] + i * g
            he = o * jnp.tanh(ce)
            new_h.append(he)
            new_c.append(ce)
            contrib = ge[e] * he
            comb = contrib if comb is None else comb + contrib
        comb_ref[sl, :] = comb
        return (tuple(new_h), tuple(new_c))

    hs_fin, cs_fin = jax.lax.fori_loop(0, _TS, step, (h0, c0), unroll=4)
    for e in range(_E):
        h_ref[e * _B:(e + 1) * _B, :] = hs_fin[e]
        c_ref[e * _B:(e + 1) * _B, :] = cs_fin[e]

    out_ref[...] = (
        jnp.dot(comb_ref[...].astype(BF16), wfc_ref[...],
                preferred_element_type=F32)
        + bfc_ref[...]
    )


def kernel(x, spk, Wg, bg, W_ih, W_hh, b_ih, b_hh, W_fc, b_fc):
    x2 = jnp.swapaxes(x, 0, 1).reshape(_S * _B, _D)
    W_ihT = jnp.swapaxes(W_ih, 1, 2).astype(BF16)
    W_hhT = jnp.swapaxes(W_hh, 1, 2).astype(BF16)
    b = (b_ih + b_hh).reshape(_E, 1, 4 * _H)
    W_fcT = W_fc.T.astype(BF16)
    bfc2 = b_fc.reshape(1, _O)
    bg2 = bg.reshape(1, _E)
    WgT = Wg.T

    gates, loss2 = pl.pallas_call(
        _gating_kernel,
        out_shape=(
            jax.ShapeDtypeStruct((_B, _E), F32),
            jax.ShapeDtypeStruct((1, 1), F32),
        ),
    )(spk, WgT, bg2)

    gcol = gates.T.reshape(_E, _B, 1)

    out2 = pl.pallas_call(
        _lstm_kernel,
        grid=(_NCHUNK,),
        in_specs=[
            pl.BlockSpec((_TS * _B, _D), lambda t: (t, 0)),
            pl.BlockSpec((_E, _D, 4 * _H), lambda t: (0, 0, 0)),
            pl.BlockSpec((_E, _H, 4 * _H), lambda t: (0, 0, 0)),
            pl.BlockSpec((_E, 1, 4 * _H), lambda t: (0, 0, 0)),
            pl.BlockSpec((_E, _B, 1), lambda t: (0, 0, 0)),
            pl.BlockSpec((_H, _O), lambda t: (0, 0)),
            pl.BlockSpec((1, _O), lambda t: (0, 0)),
        ],
        out_specs=pl.BlockSpec((_TS * _B, _O), lambda t: (t, 0)),
        out_shape=jax.ShapeDtypeStruct((_S * _B, _O), F32),
        scratch_shapes=[
            pltpu.VMEM((_E, _TS * _B, 4 * _H), F32),
            pltpu.VMEM((_TS * _B, _H), F32),
            pltpu.VMEM((_E * _B, _H), F32),
            pltpu.VMEM((_E * _B, _H), F32),
        ],
    )(x2, W_ihT, W_hhT, b, gcol, W_fcT, bfc2)

    out = out2.reshape(_S, _B, _O).swapaxes(0, 1)
    return (out, loss2[0, 0])


# speedup vs baseline: 2.4673x; 1.0263x over previous
import jax
import jax.numpy as jnp
from jax.experimental import pallas as pl
from jax.experimental.pallas import tpu as pltpu

_B, _S, _D, _H, _O, _SKP, _E = 8, 512, 256, 256, 256, 256, 8
_TS = 64
_NCHUNK = _S // _TS
F32 = jnp.float32
BF16 = jnp.bfloat16


def _gating_kernel(spk_ref, wgt_ref, bg_ref, gates_ref, loss_ref):
    logits = jnp.dot(spk_ref[...], wgt_ref[...], preferred_element_type=F32) + bg_ref[...]
    col = jax.lax.broadcasted_iota(jnp.int32, (_B, _E), 1)
    m1 = jnp.max(logits, axis=1, keepdims=True)
    i1 = jnp.min(jnp.where(logits == m1, col, _E), axis=1, keepdims=True)
    masked = jnp.where(col == i1, -jnp.inf, logits)
    m2 = jnp.max(masked, axis=1, keepdims=True)
    i2 = jnp.min(jnp.where(masked == m2, col, _E), axis=1, keepdims=True)
    e2 = jnp.exp(m2 - m1)
    denom = 1.0 + e2
    gates = jnp.where(col == i1, 1.0 / denom, jnp.where(col == i2, e2 / denom, 0.0))
    gates_ref[...] = gates
    importance = jnp.sum(gates, axis=0, keepdims=True)
    load = jnp.sum((gates > 0.0).astype(F32), axis=0, keepdims=True)

    def cv2(v):
        mean = jnp.mean(v, axis=1, keepdims=True)
        var = jnp.sum((v - mean) ** 2, axis=1, keepdims=True) / (_E - 1)
        return var / (mean * mean + 1e-10)

    loss_ref[...] = (cv2(importance) + cv2(load)) * 0.01


def _lstm_kernel(x2_ref, wih_ref, whh_ref, b_ref, gcol_ref, wfc_ref, bfc_ref,
                 out_ref, xw_ref, comb_ref, h_ref, c_ref):
    t0 = pl.program_id(0)

    xc = x2_ref[...].astype(BF16)
    for e in range(_E):
        xw_ref[e] = (
            jnp.dot(xc, wih_ref[e], preferred_element_type=F32) + b_ref[e]
        )

    @pl.when(t0 == 0)
    def _():
        h_ref[...] = jnp.zeros_like(h_ref)
        c_ref[...] = jnp.zeros_like(c_ref)

    ge = [gcol_ref[e] for e in range(_E)]

    h0 = tuple(h_ref[e * _B:(e + 1) * _B, :] for e in range(_E))
    c0 = tuple(c_ref[e * _B:(e + 1) * _B, :] for e in range(_E))

    def step(t, carry):
        hs, cs = carry
        sl = pl.ds(t * _B, _B)
        comb = None
        new_h, new_c = [], []
        for e in range(_E):
            z = xw_ref[e, sl, :] + jnp.dot(
                hs[e].astype(BF16), whh_ref[e], preferred_element_type=F32)
            i = jax.nn.sigmoid(z[:, :_H])
            f = jax.nn.sigmoid(z[:, _H:2 * _H])
            g = jnp.tanh(z[:, 2 * _H:3 * _H])
            o = jax.nn.sigmoid(z[:, 3 * _H:])
            ce = f * cs[e] + i * g
            he = o * jnp.tanh(ce)
            new_h.append(he)
            new_c.append(ce)
            contrib = ge[e] * he
            comb = contrib if comb is None else comb + contrib
        comb_ref[sl, :] = comb
        return (tuple(new_h), tuple(new_c))

    hs_fin, cs_fin = jax.lax.fori_loop(0, _TS, step, (h0, c0), unroll=8)
    for e in range(_E):
        h_ref[e * _B:(e + 1) * _B, :] = hs_fin[e]
        c_ref[e * _B:(e + 1) * _B, :] = cs_fin[e]

    out_ref[...] = (
        jnp.dot(comb_ref[...].astype(BF16), wfc_ref[...],
                preferred_element_type=F32)
        + bfc_ref[...]
    )


def kernel(x, spk, Wg, bg, W_ih, W_hh, b_ih, b_hh, W_fc, b_fc):
    x2 = jnp.swapaxes(x, 0, 1).reshape(_S * _B, _D)
    W_ihT = jnp.swapaxes(W_ih, 1, 2).astype(BF16)
    W_hhT = jnp.swapaxes(W_hh, 1, 2).astype(BF16)
    b = (b_ih + b_hh).reshape(_E, 1, 4 * _H)
    W_fcT = W_fc.T.astype(BF16)
    bfc2 = b_fc.reshape(1, _O)
    bg2 = bg.reshape(1, _E)
    WgT = Wg.T

    gates, loss2 = pl.pallas_call(
        _gating_kernel,
        out_shape=(
            jax.ShapeDtypeStruct((_B, _E), F32),
            jax.ShapeDtypeStruct((1, 1), F32),
        ),
    )(spk, WgT, bg2)

    gcol = gates.T.reshape(_E, _B, 1)

    out2 = pl.pallas_call(
        _lstm_kernel,
        grid=(_NCHUNK,),
        in_specs=[
            pl.BlockSpec((_TS * _B, _D), lambda t: (t, 0)),
            pl.BlockSpec((_E, _D, 4 * _H), lambda t: (0, 0, 0)),
            pl.BlockSpec((_E, _H, 4 * _H), lambda t: (0, 0, 0)),
            pl.BlockSpec((_E, 1, 4 * _H), lambda t: (0, 0, 0)),
            pl.BlockSpec((_E, _B, 1), lambda t: (0, 0, 0)),
            pl.BlockSpec((_H, _O), lambda t: (0, 0)),
            pl.BlockSpec((1, _O), lambda t: (0, 0)),
        ],
        out_specs=pl.BlockSpec((_TS * _B, _O), lambda t: (t, 0)),
        out_shape=jax.ShapeDtypeStruct((_S * _B, _O), F32),
        scratch_shapes=[
            pltpu.VMEM((_E, _TS * _B, 4 * _H), F32),
            pltpu.VMEM((_TS * _B, _H), F32),
            pltpu.VMEM((_E * _B, _H), F32),
            pltpu.VMEM((_E * _B, _H), F32),
        ],
    )(x2, W_ihT, W_hhT, b, gcol, W_fcT, bfc2)

    out = out2.reshape(_S, _B, _O).swapaxes(0, 1)
    return (out, loss2[0, 0])
